# baseline (device time: 271190 ns/iter reference)
import jax
import jax.numpy as jnp
from jax import lax
from jax.experimental import pallas as pl
from jax.experimental.pallas import tpu as pltpu

N_DEV = 8

FWD_SENDS = (0, 1, 2, 3, 4, 5, 6)
FWD_CREDIT_SIGNALS = (0, 1, 2)
FWD_CREDIT_WAITS = (4, 5, 6)
REV_CREDIT_SIGNALS = (0, 1, 3)
REV_CREDIT_WAITS = (4, 5, 7)


def kernel(A, B):
    m_per, k = A.shape
    k2, n = B.shape
    assert k == k2
    H = m_per // 2
    n_half = n // 2

    def body(a_ref, b_ref, out_ref, stage_ref, a16_ref, b16_ref,
             fwd_ref, rev_ref, cbuf_ref,
             fsend, frecv, rsend, rrecv,
             fwd_credit, rev_credit, copy_sems, stage_sem):
        my = lax.axis_index("i")
        left = (my - 1) % N_DEV
        right = (my + 1) % N_DEV

        def load16(src, dst):
            cp = pltpu.make_async_copy(src, stage_ref, stage_sem)
            cp.start()
            cp.wait()
            dst[...] = stage_ref[...].astype(jnp.bfloat16)

        load16(a_ref.at[pl.ds(0, H), :], a16_ref.at[pl.ds(0, H), :])
        load16(a_ref.at[pl.ds(H, H), :], a16_ref.at[pl.ds(H, H), :])

        barrier_sem = pltpu.get_barrier_semaphore()
        for nbr in (left, right):
            pl.semaphore_signal(
                barrier_sem, inc=1,
                device_id=(nbr,), device_id_type=pl.DeviceIdType.MESH,
            )
        pl.semaphore_wait(barrier_sem, 2)

        pending = [None, None]
        state = {"j": 0}

        def compute_store_half(src, origin, half):
            slot = state["j"] % 2
            state["j"] += 1
            if pending[slot] is not None:
                pending[slot].wait()
            cbuf_ref[slot] = jnp.dot(src, b16_ref[...],
                                     preferred_element_type=jnp.float32)
            cp = pltpu.make_async_copy(
                cbuf_ref.at[slot],
                out_ref.at[pl.ds(origin * m_per + half * H, H), :],
                copy_sems.at[slot])
            cp.start()
            pending[slot] = cp

        def signal(sem, nbr):
            pl.semaphore_signal(sem, inc=1, device_id=(nbr,),
                                device_id_type=pl.DeviceIdType.MESH)

        fwd_sent = {}
        rev_sent = {}

        def fwd_send(s):
            src = (a16_ref.at[pl.ds(s * H, H), :] if s < 2
                   else fwd_ref.at[(s - 2) % 4])
            r = pltpu.make_async_remote_copy(
                src_ref=src, dst_ref=fwd_ref.at[s % 4],
                send_sem=fsend.at[s % 4], recv_sem=frecv.at[s % 4],
                device_id=(right,), device_id_type=pl.DeviceIdType.MESH)
            r.start()
            fwd_sent[s] = r

        def rev_send(s):
            src = (a16_ref.at[pl.ds(s * H, H), :] if s < 2
                   else rev_ref.at[(s - 2) % 4])
            r = pltpu.make_async_remote_copy(
                src_ref=src, dst_ref=rev_ref.at[s % 4],
                send_sem=rsend.at[s % 4], recv_sem=rrecv.at[s % 4],
                device_id=(left,), device_id_type=pl.DeviceIdType.MESH)
            r.start()
            rev_sent[s] = r

        def process_fwd(e):
            fwd_sent[e].wait_recv()
            fwd_sent[e].wait_send()
            if e - 2 in FWD_CREDIT_SIGNALS:
                signal(fwd_credit, left)
            s = e + 2
            if s in FWD_SENDS:
                if s in FWD_CREDIT_WAITS:
                    pl.semaphore_wait(fwd_credit, 1)
                fwd_send(s)
            compute_store_half(fwd_ref[e % 4], (my - (e // 2) - 1) % N_DEV,
                               e % 2)

        def process_rev(e):
            fwd_map = {0: 2, 1: 3, 2: 4, 3: 5, 5: 7}
            rev_sent[e].wait_recv()
            rev_sent[e].wait_send()
            if e - 2 in REV_CREDIT_SIGNALS:
                signal(rev_credit, right)
            s = fwd_map.get(e)
            if s is not None:
                if s in REV_CREDIT_WAITS:
                    pl.semaphore_wait(rev_credit, 1)
                rev_send(s)
            compute_store_half(rev_ref[e % 4], (my + (e // 2) + 1) % N_DEV,
                               e % 2)

        fwd_send(0)
        rev_send(0)
        fwd_send(1)
        rev_send(1)
        load16(b_ref.at[:, pl.ds(0, n_half)], b16_ref.at[:, pl.ds(0, n_half)])
        load16(b_ref.at[:, pl.ds(n_half, n_half)],
               b16_ref.at[:, pl.ds(n_half, n_half)])
        compute_store_half(a16_ref[pl.ds(0, H), :], my, 0)
        compute_store_half(a16_ref[pl.ds(H, H), :], my, 1)

        for e in range(6):
            process_fwd(e)
            process_rev(e)
        process_fwd(6)
        process_rev(7)

        pending[0].wait()
        pending[1].wait()

    return pl.pallas_call(
        body,
        out_shape=jax.ShapeDtypeStruct((N_DEV * m_per, n), jnp.float32),
        in_specs=[
            pl.BlockSpec(memory_space=pl.ANY),
            pl.BlockSpec(memory_space=pl.ANY),
        ],
        out_specs=pl.BlockSpec(memory_space=pl.ANY),
        scratch_shapes=[
            pltpu.VMEM((H, k), jnp.float32),
            pltpu.VMEM((m_per, k), jnp.bfloat16),
            pltpu.VMEM((k, n), jnp.bfloat16),
            pltpu.VMEM((4, H, k), jnp.bfloat16),
            pltpu.VMEM((4, H, k), jnp.bfloat16),
            pltpu.VMEM((2, H, n), jnp.float32),
            pltpu.SemaphoreType.DMA((4,)),
            pltpu.SemaphoreType.DMA((4,)),
            pltpu.SemaphoreType.DMA((4,)),
            pltpu.SemaphoreType.DMA((4,)),
            pltpu.SemaphoreType.REGULAR,
            pltpu.SemaphoreType.REGULAR,
            pltpu.SemaphoreType.DMA((2,)),
            pltpu.SemaphoreType.DMA,
        ],
        compiler_params=pltpu.CompilerParams(
            collective_id=0,
            vmem_limit_bytes=63 * 1024 * 1024,
        ),
    )(A, B)


# device time: 270535 ns/iter; 1.0024x vs baseline; 1.0024x over previous
import jax
import jax.numpy as jnp
from jax import lax
from jax.experimental import pallas as pl
from jax.experimental.pallas import tpu as pltpu

N_DEV = 8

FWD_SENDS = (0, 1, 2, 3, 4, 5, 6)
FWD_CREDIT_SIGNALS = (0, 1, 2)
FWD_CREDIT_WAITS = (4, 5, 6)
REV_CREDIT_SIGNALS = (0, 1, 3)
REV_CREDIT_WAITS = (4, 5, 7)


def kernel(A, B):
    m_per, k = A.shape
    k2, n = B.shape
    assert k == k2
    H = m_per // 2
    n_half = n // 2

    def body(a_ref, b_ref, out_ref, stage_ref, a16_ref, b16_ref,
             fwd_ref, rev_ref, cbuf_ref,
             fsend, frecv, rsend, rrecv,
             fwd_credit, rev_credit, copy_sems, stage_sem):
        my = lax.axis_index("i")
        left = (my - 1) % N_DEV
        right = (my + 1) % N_DEV

        def load16(src, dst):
            cp = pltpu.make_async_copy(src, stage_ref, stage_sem)
            cp.start()
            cp.wait()
            dst[...] = stage_ref[...].astype(jnp.bfloat16)

        load16(a_ref.at[pl.ds(0, H), :], a16_ref.at[pl.ds(0, H), :])
        load16(a_ref.at[pl.ds(H, H), :], a16_ref.at[pl.ds(H, H), :])

        barrier_sem = pltpu.get_barrier_semaphore()
        for nbr in (left, right):
            pl.semaphore_signal(
                barrier_sem, inc=1,
                device_id=(nbr,), device_id_type=pl.DeviceIdType.MESH,
            )
        pl.semaphore_wait(barrier_sem, 2)

        pending = [None, None]
        state = {"j": 0}

        def compute_store_half(src, origin, half):
            slot = state["j"] % 2
            state["j"] += 1
            if pending[slot] is not None:
                pending[slot].wait()
            cbuf_ref[slot] = jnp.dot(src, b16_ref[...],
                                     preferred_element_type=jnp.float32)
            cp = pltpu.make_async_copy(
                cbuf_ref.at[slot],
                out_ref.at[pl.ds(origin * m_per + half * H, H), :],
                copy_sems.at[slot])
            cp.start()
            pending[slot] = cp

        def signal(sem, nbr):
            pl.semaphore_signal(sem, inc=1, device_id=(nbr,),
                                device_id_type=pl.DeviceIdType.MESH)

        fwd_sent = {}
        rev_sent = {}

        def fwd_send(s):
            src = (a16_ref.at[pl.ds(s * H, H), :] if s < 2
                   else fwd_ref.at[(s - 2) % 4])
            r = pltpu.make_async_remote_copy(
                src_ref=src, dst_ref=fwd_ref.at[s % 4],
                send_sem=fsend.at[s % 4], recv_sem=frecv.at[s % 4],
                device_id=(right,), device_id_type=pl.DeviceIdType.MESH)
            r.start()
            fwd_sent[s] = r

        def rev_send(s):
            src = (a16_ref.at[pl.ds(s * H, H), :] if s < 2
                   else rev_ref.at[(s - 2) % 4])
            r = pltpu.make_async_remote_copy(
                src_ref=src, dst_ref=rev_ref.at[s % 4],
                send_sem=rsend.at[s % 4], recv_sem=rrecv.at[s % 4],
                device_id=(left,), device_id_type=pl.DeviceIdType.MESH)
            r.start()
            rev_sent[s] = r

        def comm_fwd(e):
            fwd_sent[e].wait_recv()
            fwd_sent[e].wait_send()
            if e - 2 in FWD_CREDIT_SIGNALS:
                signal(fwd_credit, left)
            s = e + 2
            if s in FWD_SENDS:
                if s in FWD_CREDIT_WAITS:
                    pl.semaphore_wait(fwd_credit, 1)
                fwd_send(s)

        def comm_rev(e):
            fwd_map = {0: 2, 1: 3, 2: 4, 3: 5, 5: 7}
            rev_sent[e].wait_recv()
            rev_sent[e].wait_send()
            if e - 2 in REV_CREDIT_SIGNALS:
                signal(rev_credit, right)
            s = fwd_map.get(e)
            if s is not None:
                if s in REV_CREDIT_WAITS:
                    pl.semaphore_wait(rev_credit, 1)
                rev_send(s)

        def compute_fwd(e):
            compute_store_half(fwd_ref[e % 4], (my - (e // 2) - 1) % N_DEV,
                               e % 2)

        def compute_rev(e):
            compute_store_half(rev_ref[e % 4], (my + (e // 2) + 1) % N_DEV,
                               e % 2)

        fwd_send(0)
        rev_send(0)
        fwd_send(1)
        rev_send(1)
        load16(b_ref.at[:, pl.ds(0, n_half)], b16_ref.at[:, pl.ds(0, n_half)])
        load16(b_ref.at[:, pl.ds(n_half, n_half)],
               b16_ref.at[:, pl.ds(n_half, n_half)])
        compute_store_half(a16_ref[pl.ds(0, H), :], my, 0)
        compute_store_half(a16_ref[pl.ds(H, H), :], my, 1)

        for e in range(6):
            comm_fwd(e)
            comm_rev(e)
            compute_fwd(e)
            compute_rev(e)
        comm_fwd(6)
        comm_rev(7)
        compute_fwd(6)
        compute_rev(7)

        pending[0].wait()
        pending[1].wait()

    return pl.pallas_call(
        body,
        out_shape=jax.ShapeDtypeStruct((N_DEV * m_per, n), jnp.float32),
        in_specs=[
            pl.BlockSpec(memory_space=pl.ANY),
            pl.BlockSpec(memory_space=pl.ANY),
        ],
        out_specs=pl.BlockSpec(memory_space=pl.ANY),
        scratch_shapes=[
            pltpu.VMEM((H, k), jnp.float32),
            pltpu.VMEM((m_per, k), jnp.bfloat16),
            pltpu.VMEM((k, n), jnp.bfloat16),
            pltpu.VMEM((4, H, k), jnp.bfloat16),
            pltpu.VMEM((4, H, k), jnp.bfloat16),
            pltpu.VMEM((2, H, n), jnp.float32),
            pltpu.SemaphoreType.DMA((4,)),
            pltpu.SemaphoreType.DMA((4,)),
            pltpu.SemaphoreType.DMA((4,)),
            pltpu.SemaphoreType.DMA((4,)),
            pltpu.SemaphoreType.REGULAR,
            pltpu.SemaphoreType.REGULAR,
            pltpu.SemaphoreType.DMA((2,)),
            pltpu.SemaphoreType.DMA,
        ],
        compiler_params=pltpu.CompilerParams(
            collective_id=0,
            vmem_limit_bytes=63 * 1024 * 1024,
        ),
    )(A, B)


# device time: 269669 ns/iter; 1.0056x vs baseline; 1.0032x over previous
import jax
import jax.numpy as jnp
from jax import lax
from jax.experimental import pallas as pl
from jax.experimental.pallas import tpu as pltpu

N_DEV = 8

FWD_SENDS = (0, 1, 2, 3, 4, 5, 6)
FWD_CREDIT_SIGNALS = (0, 1, 2)
FWD_CREDIT_WAITS = (4, 5, 6)
REV_CREDIT_SIGNALS = (0, 1, 3)
REV_CREDIT_WAITS = (4, 5, 7)


def kernel(A, B):
    m_per, k = A.shape
    k2, n = B.shape
    assert k == k2
    H = m_per // 2
    n_half = n // 2

    def body(a_ref, b_ref, out_ref, stage_ref, a16_ref, b16_ref,
             fwd_ref, rev_ref, cbuf_ref,
             fsend, frecv, rsend, rrecv,
             fwd_credit, rev_credit, copy_sems, stage_sem):
        my = lax.axis_index("i")
        left = (my - 1) % N_DEV
        right = (my + 1) % N_DEV

        def load16(src, dst):
            cp = pltpu.make_async_copy(src, stage_ref, stage_sem)
            cp.start()
            cp.wait()
            dst[...] = stage_ref[...].astype(jnp.bfloat16)

        load16(a_ref.at[pl.ds(0, H), :], a16_ref.at[pl.ds(0, H), :])
        load16(a_ref.at[pl.ds(H, H), :], a16_ref.at[pl.ds(H, H), :])

        barrier_sem = pltpu.get_barrier_semaphore()
        for nbr in (left, right):
            pl.semaphore_signal(
                barrier_sem, inc=1,
                device_id=(nbr,), device_id_type=pl.DeviceIdType.MESH,
            )
        pl.semaphore_wait(barrier_sem, 2)

        pending = [None, None]
        state = {"j": 0}

        def compute_store_half(src, origin, half):
            slot = state["j"] % 2
            state["j"] += 1
            if pending[slot] is not None:
                pending[slot].wait()
            cbuf_ref[slot] = jnp.dot(src, b16_ref[...],
                                     preferred_element_type=jnp.float32)
            cp = pltpu.make_async_copy(
                cbuf_ref.at[slot],
                out_ref.at[pl.ds(origin * m_per + half * H, H), :],
                copy_sems.at[slot])
            cp.start()
            pending[slot] = cp

        def signal(sem, nbr):
            pl.semaphore_signal(sem, inc=1, device_id=(nbr,),
                                device_id_type=pl.DeviceIdType.MESH)

        fwd_sent = {}
        rev_sent = {}

        def fwd_send(s):
            src = (a16_ref.at[pl.ds(s * H, H), :] if s < 2
                   else fwd_ref.at[(s - 2) % 4])
            r = pltpu.make_async_remote_copy(
                src_ref=src, dst_ref=fwd_ref.at[s % 4],
                send_sem=fsend.at[s % 4], recv_sem=frecv.at[s % 4],
                device_id=(right,), device_id_type=pl.DeviceIdType.MESH)
            r.start()
            fwd_sent[s] = r

        def rev_send(s):
            src = (a16_ref.at[pl.ds(s * H, H), :] if s < 2
                   else rev_ref.at[(s - 2) % 4])
            r = pltpu.make_async_remote_copy(
                src_ref=src, dst_ref=rev_ref.at[s % 4],
                send_sem=rsend.at[s % 4], recv_sem=rrecv.at[s % 4],
                device_id=(left,), device_id_type=pl.DeviceIdType.MESH)
            r.start()
            rev_sent[s] = r

        def comm_fwd(e):
            fwd_sent[e].wait_recv()
            fwd_sent[e].wait_send()
            if e - 2 in FWD_CREDIT_SIGNALS:
                signal(fwd_credit, left)
            s = e + 2
            if s in FWD_SENDS:
                if s in FWD_CREDIT_WAITS:
                    pl.semaphore_wait(fwd_credit, 1)
                fwd_send(s)

        def comm_rev(e):
            fwd_map = {0: 2, 1: 3, 2: 4, 3: 5, 5: 7}
            rev_sent[e].wait_recv()
            rev_sent[e].wait_send()
            if e - 2 in REV_CREDIT_SIGNALS:
                signal(rev_credit, right)
            s = fwd_map.get(e)
            if s is not None:
                if s in REV_CREDIT_WAITS:
                    pl.semaphore_wait(rev_credit, 1)
                rev_send(s)

        def compute_fwd(e):
            compute_store_half(fwd_ref[e % 4], (my - (e // 2) - 1) % N_DEV,
                               e % 2)

        def compute_rev(e):
            compute_store_half(rev_ref[e % 4], (my + (e // 2) + 1) % N_DEV,
                               e % 2)

        fwd_send(0)
        rev_send(0)
        fwd_send(1)
        rev_send(1)
        load16(b_ref.at[:, pl.ds(0, n_half)], b16_ref.at[:, pl.ds(0, n_half)])
        load16(b_ref.at[:, pl.ds(n_half, n_half)],
               b16_ref.at[:, pl.ds(n_half, n_half)])
        compute_store_half(a16_ref[pl.ds(0, H), :], my, 0)
        compute_store_half(a16_ref[pl.ds(H, H), :], my, 1)

        for e in range(6):
            comm_fwd(e)
            comm_rev(e)
            compute_fwd(e)
            compute_rev(e)
        comm_fwd(6)
        comm_rev(7)
        compute_fwd(6)
        compute_rev(7)

        pending[0].wait()
        pending[1].wait()

    return pl.pallas_call(
        body,
        out_shape=jax.ShapeDtypeStruct((N_DEV * m_per, n), jnp.float32),
        in_specs=[
            pl.BlockSpec(memory_space=pl.ANY),
            pl.BlockSpec(memory_space=pl.ANY),
        ],
        out_specs=pl.BlockSpec(memory_space=pltpu.MemorySpace.HBM),
        scratch_shapes=[
            pltpu.VMEM((H, k), jnp.float32),
            pltpu.VMEM((m_per, k), jnp.bfloat16),
            pltpu.VMEM((k, n), jnp.bfloat16),
            pltpu.VMEM((4, H, k), jnp.bfloat16),
            pltpu.VMEM((4, H, k), jnp.bfloat16),
            pltpu.VMEM((2, H, n), jnp.float32),
            pltpu.SemaphoreType.DMA((4,)),
            pltpu.SemaphoreType.DMA((4,)),
            pltpu.SemaphoreType.DMA((4,)),
            pltpu.SemaphoreType.DMA((4,)),
            pltpu.SemaphoreType.REGULAR,
            pltpu.SemaphoreType.REGULAR,
            pltpu.SemaphoreType.DMA((2,)),
            pltpu.SemaphoreType.DMA,
        ],
        compiler_params=pltpu.CompilerParams(
            collective_id=0,
            vmem_limit_bytes=63 * 1024 * 1024,
        ),
    )(A, B)
